# baseline (speedup 1.0000x reference)
import jax
import jax.numpy as jnp
from jax.experimental import pallas as pl
from jax.experimental.pallas import tpu as pltpu

_OUT_LEN = 245760
_NG = (1024, 2048, 4096, 8192)
_GBASE = (0, 16 * 1024, 16 * 3072, 16 * 7168)


def _body(x_ref, memt_ref, out_ref, o0, o1, o2, o3, sem_out):
    ov = (o0, o1, o2, o3)

    out_cps = []
    for g in range(4):
        n = _NG[g]
        ov[g][:, 0 : n - 1] = memt_ref[
            pl.ds(16 * g, 16), pl.ds(8192 - n + 1, n - 1)
        ]
        ov[g][:, pl.ds(n - 1, 1)] = x_ref[pl.ds(16 * g, 16), :]
        for c in range(16):
            cp = pltpu.make_async_copy(
                ov[g].at[c, :],
                out_ref.at[pl.ds(_GBASE[g] + c * n, n)],
                sem_out,
            )
            cp.start()
            out_cps.append(cp)
    for cp in out_cps:
        cp.wait()


@jax.jit
def kernel(x, memory):
    memt = memory.T
    return pl.pallas_call(
        _body,
        out_shape=jax.ShapeDtypeStruct((_OUT_LEN,), jnp.float32),
        in_specs=[
            pl.BlockSpec(memory_space=pltpu.VMEM),
            pl.BlockSpec(memory_space=pltpu.VMEM),
        ],
        out_specs=pl.BlockSpec(memory_space=pl.ANY),
        scratch_shapes=[
            pltpu.VMEM((16, 1024), jnp.float32),
            pltpu.VMEM((16, 2048), jnp.float32),
            pltpu.VMEM((16, 4096), jnp.float32),
            pltpu.VMEM((16, 8192), jnp.float32),
            pltpu.SemaphoreType.DMA,
        ],
    )(x.reshape(64, 1), memt)


# speedup vs baseline: 1.2623x; 1.2623x over previous
import jax
import jax.numpy as jnp
from jax.experimental import pallas as pl
from jax.experimental.pallas import tpu as pltpu

_OUT_LEN = 245760
_NG = (1024, 2048, 4096, 8192)
_GBASE = (0, 16 * 1024, 16 * 3072, 16 * 7168)


def _body(x_ref, t0, t1, t2, t3, out_ref, o0, o1, o2, o3, sem_out):
    tv = (t0, t1, t2, t3)
    ov = (o0, o1, o2, o3)

    out_cps = []
    for g in range(4):
        n = _NG[g]
        ov[g][:, 0 : n - 1] = tv[g][:, 1:n]
        ov[g][:, pl.ds(n - 1, 1)] = x_ref[pl.ds(16 * g, 16), :]
        for c in range(16):
            cp = pltpu.make_async_copy(
                ov[g].at[c, :],
                out_ref.at[pl.ds(_GBASE[g] + c * n, n)],
                sem_out,
            )
            cp.start()
            out_cps.append(cp)
    for cp in out_cps:
        cp.wait()


@jax.jit
def kernel(x, memory):
    memt = memory.T
    in_specs = [pl.BlockSpec(memory_space=pltpu.VMEM)]
    for g in range(4):
        n = _NG[g]
        in_specs.append(
            pl.BlockSpec((16, n), lambda i, g=g, n=n: (g, 8192 // n - 1))
        )
    return pl.pallas_call(
        _body,
        grid=(1,),
        out_shape=jax.ShapeDtypeStruct((_OUT_LEN,), jnp.float32),
        in_specs=in_specs,
        out_specs=pl.BlockSpec(memory_space=pl.ANY),
        scratch_shapes=[
            pltpu.VMEM((16, 1024), jnp.float32),
            pltpu.VMEM((16, 2048), jnp.float32),
            pltpu.VMEM((16, 4096), jnp.float32),
            pltpu.VMEM((16, 8192), jnp.float32),
            pltpu.SemaphoreType.DMA,
        ],
    )(x.reshape(64, 1), memt, memt, memt, memt)
